# baseline (speedup 1.0000x reference)
import jax
import jax.numpy as jnp
from jax import lax
from jax.experimental import pallas as pl
from jax.experimental.pallas import tpu as pltpu
from jax.experimental.pallas import tpu_sc as plsc

CAP = 100000
D = 128
B = 16384

NC = 2
NS = 16
NW = NC * NS
L = 16

W_MAIN = 3128
W_LAST = CAP - (NW - 1) * W_MAIN
ROWS_PAD = 3136
MCAP = 2048
CHUNK = 128


def _sc_body(mem_hbm, idx_hbm, val_hbm, out_hbm,
             idx_v, winner_v, mpos_v, mrow_v, fpos_v, frow_v,
             rowbuf_v, copy_sem, dma_sem):
    wid = lax.axis_index("s") * NC + lax.axis_index("c")
    lo = wid * W_MAIN
    size = jnp.where(wid == NW - 1, W_LAST, W_MAIN)

    lane = lax.iota(jnp.int32, L)
    neg1 = jnp.full((L,), -1, jnp.int32)
    ones = jnp.full((L,), 1, jnp.int32)
    zeros = jnp.full((L,), 0, jnp.int32)
    lov = jnp.full((L,), lo, jnp.int32)
    szv = jnp.full((L,), size, jnp.int32)

    @pl.when(wid != NW - 1)
    def _():
        pltpu.make_async_copy(
            mem_hbm.at[pl.ds(lo, W_MAIN)],
            out_hbm.at[pl.ds(lo, W_MAIN)],
            copy_sem,
        ).start()

    @pl.when(wid == NW - 1)
    def _():
        pltpu.make_async_copy(
            mem_hbm.at[pl.ds((NW - 1) * W_MAIN, W_LAST)],
            out_hbm.at[pl.ds((NW - 1) * W_MAIN, W_LAST)],
            copy_sem,
        ).start()

    pltpu.sync_copy(idx_hbm, idx_v)

    def prefix_sum_excl(mi):
        pref = mi
        for s in (1, 2, 4, 8):
            src_lane = jnp.maximum(lane - s, zeros)
            shifted = pref.at[src_lane].get(mode="promise_in_bounds")
            pref = pref + jnp.where(lane >= s, shifted, zeros)
        return pref - mi

    def filt(i, cnt):
        v = idx_v[pl.ds(i * L, L)]
        local = v - lov
        m = (local >= 0) & (local < szv)
        mi = jnp.where(m, ones, zeros)
        dest = cnt + prefix_sum_excl(mi)
        plsc.store_scatter(mpos_v, [dest], i * L + lane, mask=m)
        plsc.store_scatter(mrow_v, [dest], local, mask=m)
        pc = plsc.all_reduce_population_count(m)
        return cnt + pc[0]

    nmatch = lax.fori_loop(0, B // L, filt, jnp.int32(0))

    mpos_v[pl.ds(nmatch, L)] = neg1
    mrow_v[pl.ds(nmatch, L)] = neg1

    def winit(i, _):
        winner_v[pl.ds(i * L, L)] = neg1
        return 0

    lax.fori_loop(0, ROWS_PAD // L, winit, 0)

    def wset(g, _):
        rows = mrow_v[pl.ds(g * L, L)]
        poss = mpos_v[pl.ds(g * L, L)]
        valid = rows >= 0
        for k in range(L):
            plsc.store_scatter(winner_v, [rows], poss,
                               mask=valid & (lane == k))
        return 0

    lax.fori_loop(0, (nmatch + L - 1) // L, wset, 0)

    def compact(i, cnt):
        w = winner_v[pl.ds(i * L, L)]
        m = w >= 0
        mi = jnp.where(m, ones, zeros)
        dest = cnt + prefix_sum_excl(mi)
        plsc.store_scatter(fpos_v, [dest], w, mask=m)
        plsc.store_scatter(frow_v, [dest], lo + i * L + lane, mask=m)
        pc = plsc.all_reduce_population_count(m)
        return cnt + pc[0]

    nfinal = lax.fori_loop(0, ROWS_PAD // L, compact, jnp.int32(0))

    def tailfill(t, _):
        fpos_v[pl.ds(nfinal + t * L, L)] = neg1
        frow_v[pl.ds(nfinal + t * L, L)] = neg1
        return 0

    lax.fori_loop(0, CHUNK // L, tailfill, 0)

    @pl.when(wid != NW - 1)
    def _():
        pltpu.make_async_copy(
            mem_hbm.at[pl.ds(lo, W_MAIN)],
            out_hbm.at[pl.ds(lo, W_MAIN)],
            copy_sem,
        ).wait()

    @pl.when(wid == NW - 1)
    def _():
        pltpu.make_async_copy(
            mem_hbm.at[pl.ds((NW - 1) * W_MAIN, W_LAST)],
            out_hbm.at[pl.ds((NW - 1) * W_MAIN, W_LAST)],
            copy_sem,
        ).wait()

    nch = (nfinal + CHUNK - 1) // CHUNK

    def scat(c, _):
        gpos = plsc.Indices(fpos_v.at[pl.ds(c * CHUNK, CHUNK)],
                            ignored_value=-1)
        pltpu.async_copy(val_hbm.at[gpos], rowbuf_v, dma_sem).wait()
        grow = plsc.Indices(frow_v.at[pl.ds(c * CHUNK, CHUNK)],
                            ignored_value=-1)
        pltpu.async_copy(rowbuf_v, out_hbm.at[grow], dma_sem).wait()
        return 0

    lax.fori_loop(0, nch, scat, 0)


@jax.jit
def _scatter_sc(mem, idx, val):
    mesh = plsc.VectorSubcoreMesh(
        core_axis_name="c", subcore_axis_name="s",
        num_cores=NC, num_subcores=NS,
    )
    return pl.kernel(
        _sc_body,
        out_type=jax.ShapeDtypeStruct((CAP, D), jnp.float32),
        mesh=mesh,
        compiler_params=pltpu.CompilerParams(needs_layout_passes=False),
        scratch_types=[
            pltpu.VMEM((B,), jnp.int32),
            pltpu.VMEM((ROWS_PAD,), jnp.int32),
            pltpu.VMEM((MCAP,), jnp.int32),
            pltpu.VMEM((MCAP,), jnp.int32),
            pltpu.VMEM((MCAP + CHUNK,), jnp.int32),
            pltpu.VMEM((MCAP + CHUNK,), jnp.int32),
            pltpu.VMEM((CHUNK, D), jnp.float32),
            pltpu.SemaphoreType.DMA,
            pltpu.SemaphoreType.DMA,
        ],
    )(mem, idx, val)


def kernel(mem, idx, val):
    return _scatter_sc(mem, idx, val)
